# baseline (speedup 1.0000x reference)
import functools

import jax
import jax.numpy as jnp
from jax import lax
from jax.experimental import pallas as pl
from jax.experimental.pallas import tpu as pltpu
from jax.experimental.pallas import tpu_sc as plsc

_N = 256
_D = 512
_E = 16384
_F32 = jnp.float32
_BF16 = jnp.bfloat16
_I32 = jnp.int32

_NEG = 8
_NDG = 4
_EPW = _E // _NEG
_DG = _D // _NDG
_CH = 256
_NCH = _EPW // _CH


def _dot3(a, b):
    a_hi = a.astype(_BF16)
    a_lo = (a - a_hi.astype(_F32)).astype(_BF16)
    b_hi = b.astype(_BF16)
    b_lo = (b - b_hi.astype(_F32)).astype(_BF16)
    d = jnp.dot(a_hi, b_hi, preferred_element_type=_F32)
    d += jnp.dot(a_hi, b_lo, preferred_element_type=_F32)
    d += jnp.dot(a_lo, b_hi, preferred_element_type=_F32)
    return d


def _gram3(a):
    dn = (((1,), (1,)), ((), ()))
    a_hi = a.astype(_BF16)
    a_lo = (a - a_hi.astype(_F32)).astype(_BF16)
    g = lax.dot_general(a_hi, a_hi, dn, preferred_element_type=_F32)
    g += lax.dot_general(a_hi, a_lo, dn, preferred_element_type=_F32)
    g += lax.dot_general(a_lo, a_hi, dn, preferred_element_type=_F32)
    return g


def _prep_body(obj_ref, ws_w_ref, ws_b_ref, wo_w_ref, wo_b_ref, w_w_ref,
               conv_w_ref, conv_b_ref, ln1_g_ref, ln1_b_ref,
               s_ref, o_ref, conv_ref, omega_ref):
    obj = obj_ref[...]
    s = jnp.dot(obj, ws_w_ref[...], preferred_element_type=_F32)
    s_ref[...] = (s + ws_b_ref[...]) * w_w_ref[...]
    o = jnp.dot(obj, wo_w_ref[...], preferred_element_type=_F32)
    o_ref[...] = o + wo_b_ref[...]

    g = _gram3(obj)
    rows = lax.broadcasted_iota(_I32, (_N, _N), 0)
    cols = lax.broadcasted_iota(_I32, (_N, _N), 1)
    eye = (rows == cols).astype(_F32)
    diag_col = jnp.sum(g * eye, axis=1, keepdims=True)
    diag_row = jnp.sum(g * eye, axis=0, keepdims=True)
    n2 = jnp.maximum(diag_col + diag_row - 2.0 * g, 0.0)
    omega = jnp.where(n2 < 0.25, 4.0, 0.0)
    omega = jnp.where((n2 >= 0.25) & (n2 < 1.0),
                      1.0 / jnp.maximum(n2, 1e-10), omega)
    omega_ref[...] = jnp.where(rows == cols, 0.0, omega)

    mu = jnp.mean(obj, axis=1, keepdims=True)
    xc = obj - mu
    var = jnp.mean(xc * xc, axis=1, keepdims=True)
    xn = xc / jnp.sqrt(var + 1e-5) * ln1_g_ref[...] + ln1_b_ref[...]
    conv_ref[...] = jax.nn.relu(_dot3(xn, conv_w_ref[...]) + conv_b_ref[...])


_sc_mesh = plsc.VectorSubcoreMesh(core_axis_name="c", subcore_axis_name="s")


@functools.partial(
    pl.kernel,
    out_type=jax.ShapeDtypeStruct((2, _N * _N), _F32),
    mesh=_sc_mesh,
    scratch_types=[
        pltpu.VMEM((_N, _DG), _F32),
        pltpu.VMEM((_N, _DG), _F32),
        pltpu.VMEM((_CH, _DG), _F32),
        pltpu.VMEM((_EPW,), _I32),
        pltpu.VMEM((_EPW,), _I32),
        pltpu.VMEM((_EPW,), _I32),
        pltpu.VMEM((_EPW,), _F32),
        pltpu.VMEM((16,), _F32),
        pltpu.VMEM_SHARED((_N * _N,), _F32),
    ],
    compiler_params=pltpu.CompilerParams(needs_layout_passes=False),
)
def _sc_edge(s_hbm, o_hbm, phr_hbm, src_hbm, dst_hbm, wb_hbm, zeros_hbm,
             out_hbm, s_v, o_v, phr_v, src_v, dst_v, idx_v, val_v, wb_v,
             map_sh):
    c = lax.axis_index("c")
    s = lax.axis_index("s")
    w = s * 2 + c
    eg = w // _NDG
    dg = w % _NDG
    d0 = dg * _DG
    e0 = eg * _EPW

    pltpu.sync_copy(s_hbm.at[:, pl.ds(d0, _DG)], s_v)
    pltpu.sync_copy(o_hbm.at[:, pl.ds(d0, _DG)], o_v)
    pltpu.sync_copy(src_hbm.at[pl.ds(e0, _EPW)], src_v)
    pltpu.sync_copy(dst_hbm.at[pl.ds(e0, _EPW)], dst_v)
    pltpu.sync_copy(wb_hbm, wb_v)

    @pl.when(s == 0)
    def _():
        pltpu.sync_copy(zeros_hbm, map_sh)

    plsc.subcore_barrier()

    lanes = lax.iota(_I32, 16)
    wb = wb_v[...]
    base_acc = jnp.where(dg == 0, wb, jnp.zeros((16,), _F32))

    def _chunk(ch, carry):
        pltpu.sync_copy(
            phr_hbm.at[pl.ds(e0 + ch * _CH, _CH), pl.ds(d0, _DG)], phr_v)

        def _group(g, carry2):
            base = ch * _CH + g * 16
            src_g = src_v[pl.ds(base, 16)]
            dst_g = dst_v[pl.ds(base, 16)]
            lrow = lanes + g * 16

            def _dstep(dd, acc):
                for k in range(8):
                    col = jnp.full((16,), dd * 8 + k, _I32)
                    sv = plsc.load_gather(s_v, [src_g, col])
                    ov = plsc.load_gather(o_v, [dst_g, col])
                    pv = plsc.load_gather(phr_v, [lrow, col])
                    acc = acc + sv * ov * pv
                return acc

            acc = lax.fori_loop(0, _DG // 8, _dstep, base_acc)
            val_v[pl.ds(base, 16)] = acc
            idx_v[pl.ds(base, 16)] = src_g * _N + dst_g
            return carry2

        return lax.fori_loop(0, _CH // 16, _group, carry)

    lax.fori_loop(0, _NCH, _chunk, 0)

    pltpu.sync_copy(val_v, map_sh.at[idx_v], add=True)
    plsc.subcore_barrier()

    @pl.when(s == 0)
    def _():
        pltpu.sync_copy(map_sh, out_hbm.at[c])


def _finish_body(maps_ref, omega_ref, conv_ref, obj_ref, ln2_g_ref, ln2_b_ref,
                 t1w_ref, t1b_ref, t2w_ref, t2b_ref, out_ref):
    atten = maps_ref[:_N, :] + maps_ref[_N:, :]
    rows = lax.broadcasted_iota(_I32, (_N, _N), 0)
    cols = lax.broadcasted_iota(_I32, (_N, _N), 1)
    a = atten - jnp.where(rows == cols, 10000.0, 0.0)
    m = jnp.max(a, axis=1, keepdims=True)
    ex = jnp.exp(a - m)
    sm = ex / jnp.sum(ex, axis=1, keepdims=True)
    am = omega_ref[...] * sm
    context = _dot3(am, conv_ref[...])
    outputs = obj_ref[...] + context
    mu = jnp.mean(outputs, axis=1, keepdims=True)
    xc = outputs - mu
    var = jnp.mean(xc * xc, axis=1, keepdims=True)
    xn = xc / jnp.sqrt(var + 1e-5) * ln2_g_ref[...] + ln2_b_ref[...]
    h = jax.nn.relu(_dot3(xn, t1w_ref[...]) + t1b_ref[...])
    trans = _dot3(h, t2w_ref[...]) + t2b_ref[...]
    out_ref[...] = jax.nn.relu(outputs + trans)


def _full(shape):
    return pl.BlockSpec(shape, lambda *_: tuple(0 for _ in shape))


def kernel(obj_feats, phr_feats, pair_idxs, ws_w, ws_b, wo_w, wo_b, w_w, w_b,
           conv_w, conv_b, ln1_g, ln1_b, ln2_g, ln2_b,
           trans1_w, trans1_b, trans2_w, trans2_b):
    s_mat, o_mat, conv_out, omega = pl.pallas_call(
        _prep_body,
        grid=(1,),
        in_specs=[
            _full((_N, _D)), _full((_D, _D)), _full((1, _D)),
            _full((_D, _D)), _full((1, _D)), _full((1, _D)),
            _full((_D, _D)), _full((1, _D)), _full((1, _D)), _full((1, _D)),
        ],
        out_specs=[
            _full((_N, _D)), _full((_N, _D)), _full((_N, _D)), _full((_N, _N)),
        ],
        out_shape=[
            jax.ShapeDtypeStruct((_N, _D), _F32),
            jax.ShapeDtypeStruct((_N, _D), _F32),
            jax.ShapeDtypeStruct((_N, _D), _F32),
            jax.ShapeDtypeStruct((_N, _N), _F32),
        ],
    )(obj_feats, ws_w, ws_b.reshape(1, _D), wo_w, wo_b.reshape(1, _D),
      w_w.reshape(1, _D), conv_w, conv_b.reshape(1, _D),
      ln1_g.reshape(1, _D), ln1_b.reshape(1, _D))

    src = pair_idxs[:, 0]
    dst = pair_idxs[:, 1]
    wb16 = jnp.full((16,), w_b[0], _F32)
    zeros_map = jnp.zeros((_N * _N,), _F32)
    maps = _sc_edge(s_mat, o_mat, phr_feats, src, dst, wb16, zeros_map)

    return pl.pallas_call(
        _finish_body,
        grid=(1,),
        in_specs=[
            _full((2 * _N, _N)), _full((_N, _N)), _full((_N, _D)),
            _full((_N, _D)), _full((1, _D)), _full((1, _D)),
            _full((_D, 2 * _D)), _full((1, 2 * _D)),
            _full((2 * _D, _D)), _full((1, _D)),
        ],
        out_specs=_full((_N, _D)),
        out_shape=jax.ShapeDtypeStruct((_N, _D), _F32),
    )(maps.reshape(2 * _N, _N), omega, conv_out, obj_feats,
      ln2_g.reshape(1, _D), ln2_b.reshape(1, _D), trans1_w,
      trans1_b.reshape(1, 2 * _D), trans2_w, trans2_b.reshape(1, _D))


# speedup vs baseline: 7.2113x; 7.2113x over previous
import functools

import jax
import jax.numpy as jnp
from jax import lax
from jax.experimental import pallas as pl
from jax.experimental.pallas import tpu as pltpu
from jax.experimental.pallas import tpu_sc as plsc

_N = 256
_D = 512
_E = 16384
_EB = 2048
_NTC = 6
_F32 = jnp.float32
_BF16 = jnp.bfloat16
_I32 = jnp.int32

_E_SC0 = _NTC * _EB
_NEG = 8
_NDG = 4
_EPW = (_E - _E_SC0) // _NEG
_DG = _D // _NDG
_CH = 128
_NCH = _EPW // _CH


def _dot3(a, b):
    a_hi = a.astype(_BF16)
    a_lo = (a - a_hi.astype(_F32)).astype(_BF16)
    b_hi = b.astype(_BF16)
    b_lo = (b - b_hi.astype(_F32)).astype(_BF16)
    d = jnp.dot(a_hi, b_hi, preferred_element_type=_F32)
    d += jnp.dot(a_hi, b_lo, preferred_element_type=_F32)
    d += jnp.dot(a_lo, b_hi, preferred_element_type=_F32)
    return d


def _gram3(a):
    dn = (((1,), (1,)), ((), ()))
    a_hi = a.astype(_BF16)
    a_lo = (a - a_hi.astype(_F32)).astype(_BF16)
    g = lax.dot_general(a_hi, a_hi, dn, preferred_element_type=_F32)
    g += lax.dot_general(a_hi, a_lo, dn, preferred_element_type=_F32)
    g += lax.dot_general(a_lo, a_hi, dn, preferred_element_type=_F32)
    return g


def _prep_body(obj_ref, ws_w_ref, ws_b_ref, wo_w_ref, wo_b_ref, w_w_ref,
               conv_w_ref, conv_b_ref, ln1_g_ref, ln1_b_ref,
               s_ref, o_ref, sbf_ref, obf_ref, conv_ref, omega_ref):
    obj = obj_ref[...]
    s = jnp.dot(obj, ws_w_ref[...], preferred_element_type=_F32)
    s_mod = (s + ws_b_ref[...]) * w_w_ref[...]
    o = jnp.dot(obj, wo_w_ref[...], preferred_element_type=_F32)
    o_mod = o + wo_b_ref[...]
    s_ref[...] = s_mod
    o_ref[...] = o_mod
    sbf_ref[...] = s_mod.astype(_BF16)
    obf_ref[...] = o_mod.astype(_BF16)

    g = _gram3(obj)
    rows = lax.broadcasted_iota(_I32, (_N, _N), 0)
    cols = lax.broadcasted_iota(_I32, (_N, _N), 1)
    eye = (rows == cols).astype(_F32)
    diag_col = jnp.sum(g * eye, axis=1, keepdims=True)
    diag_row = jnp.sum(g * eye, axis=0, keepdims=True)
    n2 = jnp.maximum(diag_col + diag_row - 2.0 * g, 0.0)
    omega = jnp.where(n2 < 0.25, 4.0, 0.0)
    omega = jnp.where((n2 >= 0.25) & (n2 < 1.0),
                      1.0 / jnp.maximum(n2, 1e-10), omega)
    omega_ref[...] = jnp.where(rows == cols, 0.0, omega)

    mu = jnp.mean(obj, axis=1, keepdims=True)
    xc = obj - mu
    var = jnp.mean(xc * xc, axis=1, keepdims=True)
    xn = xc / jnp.sqrt(var + 1e-5) * ln1_g_ref[...] + ln1_b_ref[...]
    conv_ref[...] = jax.nn.relu(_dot3(xn, conv_w_ref[...]) + conv_b_ref[...])


def _tc_edge_body(src_ref, dst_ref, phr_ref, s_ref, o_ref, wb_ref, atten_ref):
    src = src_ref[0, 0, :]
    dst = dst_ref[0, 0, :]
    ids = lax.broadcasted_iota(_I32, (_EB, _N), 1)
    oh_s = (src[:, None] == ids).astype(_BF16)
    oh_d = (dst[:, None] == ids).astype(_BF16)
    gs = jnp.dot(oh_s, s_ref[...], preferred_element_type=_F32)
    go = jnp.dot(oh_d, o_ref[...], preferred_element_type=_F32)
    t = gs * go * phr_ref[...]
    af = jnp.sum(t, axis=1) + wb_ref[0, 0]
    weighted = oh_s * af[:, None].astype(_BF16)
    contrib = lax.dot_general(weighted, oh_d, (((0,), (0,)), ((), ())),
                              preferred_element_type=_F32)

    @pl.when(pl.program_id(0) == 0)
    def _():
        atten_ref[...] = jnp.zeros_like(atten_ref)

    atten_ref[...] += contrib


_sc_mesh = plsc.VectorSubcoreMesh(core_axis_name="c", subcore_axis_name="s")


@functools.partial(
    pl.kernel,
    out_type=jax.ShapeDtypeStruct((2, _N * _N), _F32),
    mesh=_sc_mesh,
    scratch_types=[
        pltpu.VMEM((_N, _DG), _F32),
        pltpu.VMEM((_N, _DG), _F32),
        pltpu.VMEM((_CH, _DG), _F32),
        pltpu.VMEM((_CH, _DG), _F32),
        pltpu.SemaphoreType.DMA,
        pltpu.SemaphoreType.DMA,
        pltpu.VMEM((_EPW,), _I32),
        pltpu.VMEM((_EPW,), _I32),
        pltpu.VMEM((_EPW,), _I32),
        pltpu.VMEM((_EPW,), _F32),
        pltpu.VMEM((16,), _F32),
        pltpu.VMEM_SHARED((_N * _N,), _F32),
    ],
    compiler_params=pltpu.CompilerParams(needs_layout_passes=False),
)
def _sc_edge(s_hbm, o_hbm, phr_hbm, src_hbm, dst_hbm, wb_hbm, zeros_hbm,
             out_hbm, s_v, o_v, phr_v0, phr_v1, sem0, sem1, src_v, dst_v,
             idx_v, val_v, wb_v, map_sh):
    c = lax.axis_index("c")
    s = lax.axis_index("s")
    w = s * 2 + c
    eg = w // _NDG
    dg = w % _NDG
    d0 = dg * _DG
    e0 = _E_SC0 + eg * _EPW

    pltpu.sync_copy(s_hbm.at[:, pl.ds(d0, _DG)], s_v)
    pltpu.sync_copy(o_hbm.at[:, pl.ds(d0, _DG)], o_v)
    pltpu.sync_copy(src_hbm.at[pl.ds(e0, _EPW)], src_v)
    pltpu.sync_copy(dst_hbm.at[pl.ds(e0, _EPW)], dst_v)
    pltpu.sync_copy(wb_hbm, wb_v)

    @pl.when(s == 0)
    def _():
        pltpu.sync_copy(zeros_hbm, map_sh)

    plsc.subcore_barrier()

    lanes = lax.iota(_I32, 16)
    wb = wb_v[...]
    base_acc = jnp.where(dg == 0, wb, jnp.zeros((16,), _F32))

    def _phr_start(ch, buf, sem):
        pltpu.async_copy(
            phr_hbm.at[pl.ds(e0 + ch * _CH, _CH), pl.ds(d0, _DG)], buf, sem)

    def _phr_wait(buf, sem):
        pltpu.make_async_copy(phr_hbm.at[pl.ds(e0, _CH), pl.ds(d0, _DG)],
                              buf, sem).wait()

    def _do_chunk(ch, phr_v):
        def _group(g, carry2):
            base = ch * _CH + g * 16
            src_g = src_v[pl.ds(base, 16)]
            dst_g = dst_v[pl.ds(base, 16)]
            lrow = lanes + g * 16

            def _dstep(dd, acc):
                prods = []
                for k in range(8):
                    col = (jnp.full((16,), dd * 8 + k, _I32) + lanes) & (_DG - 1)
                    sv = plsc.load_gather(s_v, [src_g, col])
                    ov = plsc.load_gather(o_v, [dst_g, col])
                    pv = plsc.load_gather(phr_v, [lrow, col])
                    prods.append(sv * ov * pv)
                p01 = prods[0] + prods[1]
                p23 = prods[2] + prods[3]
                p45 = prods[4] + prods[5]
                p67 = prods[6] + prods[7]
                return acc + ((p01 + p23) + (p45 + p67))

            acc = lax.fori_loop(0, _DG // 8, _dstep, base_acc)
            val_v[pl.ds(base, 16)] = acc
            idx_v[pl.ds(base, 16)] = src_g * _N + dst_g
            return carry2

        lax.fori_loop(0, _CH // 16, _group, 0)

    _phr_start(0, phr_v0, sem0)

    def _pair(j, carry):
        ch = j * 2
        _phr_start(ch + 1, phr_v1, sem1)
        _phr_wait(phr_v0, sem0)
        _do_chunk(ch, phr_v0)

        @pl.when(j < _NCH // 2 - 1)
        def _():
            _phr_start(ch + 2, phr_v0, sem0)

        _phr_wait(phr_v1, sem1)
        _do_chunk(ch + 1, phr_v1)
        return carry

    lax.fori_loop(0, _NCH // 2, _pair, 0)

    pltpu.sync_copy(val_v, map_sh.at[idx_v], add=True)
    plsc.subcore_barrier()

    @pl.when(s == 0)
    def _():
        pltpu.sync_copy(map_sh, out_hbm.at[c])


def _finish_body(atten_ref, maps_ref, omega_ref, conv_ref, obj_ref,
                 ln2_g_ref, ln2_b_ref, t1w_ref, t1b_ref, t2w_ref, t2b_ref,
                 out_ref):
    atten = atten_ref[...] + maps_ref[:_N, :] + maps_ref[_N:, :]
    rows = lax.broadcasted_iota(_I32, (_N, _N), 0)
    cols = lax.broadcasted_iota(_I32, (_N, _N), 1)
    a = atten - jnp.where(rows == cols, 10000.0, 0.0)
    m = jnp.max(a, axis=1, keepdims=True)
    ex = jnp.exp(a - m)
    sm = ex / jnp.sum(ex, axis=1, keepdims=True)
    am = omega_ref[...] * sm
    context = _dot3(am, conv_ref[...])
    outputs = obj_ref[...] + context
    mu = jnp.mean(outputs, axis=1, keepdims=True)
    xc = outputs - mu
    var = jnp.mean(xc * xc, axis=1, keepdims=True)
    xn = xc / jnp.sqrt(var + 1e-5) * ln2_g_ref[...] + ln2_b_ref[...]
    h = jax.nn.relu(_dot3(xn, t1w_ref[...]) + t1b_ref[...])
    trans = _dot3(h, t2w_ref[...]) + t2b_ref[...]
    out_ref[...] = jax.nn.relu(outputs + trans)


def _full(shape):
    return pl.BlockSpec(shape, lambda *_: tuple(0 for _ in shape))


def kernel(obj_feats, phr_feats, pair_idxs, ws_w, ws_b, wo_w, wo_b, w_w, w_b,
           conv_w, conv_b, ln1_g, ln1_b, ln2_g, ln2_b,
           trans1_w, trans1_b, trans2_w, trans2_b):
    s_mat, o_mat, s_bf, o_bf, conv_out, omega = pl.pallas_call(
        _prep_body,
        grid=(1,),
        in_specs=[
            _full((_N, _D)), _full((_D, _D)), _full((1, _D)),
            _full((_D, _D)), _full((1, _D)), _full((1, _D)),
            _full((_D, _D)), _full((1, _D)), _full((1, _D)), _full((1, _D)),
        ],
        out_specs=[
            _full((_N, _D)), _full((_N, _D)), _full((_N, _D)),
            _full((_N, _D)), _full((_N, _D)), _full((_N, _N)),
        ],
        out_shape=[
            jax.ShapeDtypeStruct((_N, _D), _F32),
            jax.ShapeDtypeStruct((_N, _D), _F32),
            jax.ShapeDtypeStruct((_N, _D), _BF16),
            jax.ShapeDtypeStruct((_N, _D), _BF16),
            jax.ShapeDtypeStruct((_N, _D), _F32),
            jax.ShapeDtypeStruct((_N, _N), _F32),
        ],
    )(obj_feats, ws_w, ws_b.reshape(1, _D), wo_w, wo_b.reshape(1, _D),
      w_w.reshape(1, _D), conv_w, conv_b.reshape(1, _D),
      ln1_g.reshape(1, _D), ln1_b.reshape(1, _D))

    src = pair_idxs[:, 0]
    dst = pair_idxs[:, 1]
    wb16 = jnp.full((16,), w_b[0], _F32)
    zeros_map = jnp.zeros((_N * _N,), _F32)
    maps = _sc_edge(s_mat, o_mat, phr_feats, src, dst, wb16, zeros_map)

    src3 = src[:_E_SC0].reshape(_NTC, 1, _EB)
    dst3 = dst[:_E_SC0].reshape(_NTC, 1, _EB)
    atten_tc = pl.pallas_call(
        _tc_edge_body,
        grid=(_NTC,),
        in_specs=[
            pl.BlockSpec((1, 1, _EB), lambda i: (i, 0, 0)),
            pl.BlockSpec((1, 1, _EB), lambda i: (i, 0, 0)),
            pl.BlockSpec((_EB, _D), lambda i: (i, 0)),
            pl.BlockSpec((_N, _D), lambda i: (0, 0)),
            pl.BlockSpec((_N, _D), lambda i: (0, 0)),
            pl.BlockSpec(memory_space=pltpu.SMEM),
        ],
        out_specs=pl.BlockSpec((_N, _N), lambda i: (0, 0)),
        out_shape=jax.ShapeDtypeStruct((_N, _N), _F32),
    )(src3, dst3, phr_feats, s_bf, o_bf, w_b.reshape(1, 1))

    return pl.pallas_call(
        _finish_body,
        grid=(1,),
        in_specs=[
            _full((_N, _N)), _full((2 * _N, _N)), _full((_N, _N)),
            _full((_N, _D)), _full((_N, _D)), _full((1, _D)), _full((1, _D)),
            _full((_D, 2 * _D)), _full((1, 2 * _D)),
            _full((2 * _D, _D)), _full((1, _D)),
        ],
        out_specs=_full((_N, _D)),
        out_shape=jax.ShapeDtypeStruct((_N, _D), _F32),
    )(atten_tc, maps.reshape(2 * _N, _N), omega, conv_out, obj_feats,
      ln2_g.reshape(1, _D), ln2_b.reshape(1, _D), trans1_w,
      trans1_b.reshape(1, 2 * _D), trans2_w, trans2_b.reshape(1, _D))


# speedup vs baseline: 7.3539x; 1.0198x over previous
import functools

import jax
import jax.numpy as jnp
from jax import lax
from jax.experimental import pallas as pl
from jax.experimental.pallas import tpu as pltpu
from jax.experimental.pallas import tpu_sc as plsc

_N = 256
_D = 512
_E = 16384
_EB = 2048
_NTC = 7
_F32 = jnp.float32
_BF16 = jnp.bfloat16
_I32 = jnp.int32

_E_SC0 = _NTC * _EB
_NEG = 8
_NDG = 4
_EPW = (_E - _E_SC0) // _NEG
_DG = _D // _NDG
_CH = 128
_NCH = _EPW // _CH


def _dot3(a, b):
    a_hi = a.astype(_BF16)
    a_lo = (a - a_hi.astype(_F32)).astype(_BF16)
    b_hi = b.astype(_BF16)
    b_lo = (b - b_hi.astype(_F32)).astype(_BF16)
    d = jnp.dot(a_hi, b_hi, preferred_element_type=_F32)
    d += jnp.dot(a_hi, b_lo, preferred_element_type=_F32)
    d += jnp.dot(a_lo, b_hi, preferred_element_type=_F32)
    return d


def _gram3(a):
    dn = (((1,), (1,)), ((), ()))
    a_hi = a.astype(_BF16)
    a_lo = (a - a_hi.astype(_F32)).astype(_BF16)
    g = lax.dot_general(a_hi, a_hi, dn, preferred_element_type=_F32)
    g += lax.dot_general(a_hi, a_lo, dn, preferred_element_type=_F32)
    g += lax.dot_general(a_lo, a_hi, dn, preferred_element_type=_F32)
    return g


def _prep_body(obj_ref, ws_w_ref, ws_b_ref, wo_w_ref, wo_b_ref, w_w_ref,
               conv_w_ref, conv_b_ref, ln1_g_ref, ln1_b_ref,
               s_ref, o_ref, sbf_ref, obf_ref, conv_ref, omega_ref):
    obj = obj_ref[...]
    s = jnp.dot(obj, ws_w_ref[...], preferred_element_type=_F32)
    s_mod = (s + ws_b_ref[...]) * w_w_ref[...]
    o = jnp.dot(obj, wo_w_ref[...], preferred_element_type=_F32)
    o_mod = o + wo_b_ref[...]
    s_ref[...] = s_mod
    o_ref[...] = o_mod
    sbf_ref[...] = s_mod.astype(_BF16)
    obf_ref[...] = o_mod.astype(_BF16)

    g = _gram3(obj)
    rows = lax.broadcasted_iota(_I32, (_N, _N), 0)
    cols = lax.broadcasted_iota(_I32, (_N, _N), 1)
    eye = (rows == cols).astype(_F32)
    diag_col = jnp.sum(g * eye, axis=1, keepdims=True)
    diag_row = jnp.sum(g * eye, axis=0, keepdims=True)
    n2 = jnp.maximum(diag_col + diag_row - 2.0 * g, 0.0)
    omega = jnp.where(n2 < 0.25, 4.0, 0.0)
    omega = jnp.where((n2 >= 0.25) & (n2 < 1.0),
                      1.0 / jnp.maximum(n2, 1e-10), omega)
    omega_ref[...] = jnp.where(rows == cols, 0.0, omega)

    mu = jnp.mean(obj, axis=1, keepdims=True)
    xc = obj - mu
    var = jnp.mean(xc * xc, axis=1, keepdims=True)
    xn = xc / jnp.sqrt(var + 1e-5) * ln1_g_ref[...] + ln1_b_ref[...]
    conv_ref[...] = jax.nn.relu(_dot3(xn, conv_w_ref[...]) + conv_b_ref[...])


def _tc_edge_body(src_ref, dst_ref, phr_ref, s_ref, o_ref, wb_ref, atten_ref):
    src = src_ref[0, 0, :]
    dst = dst_ref[0, 0, :]
    ids = lax.broadcasted_iota(_I32, (_EB, _N), 1)
    oh_s = (src[:, None] == ids).astype(_BF16)
    oh_d = (dst[:, None] == ids).astype(_BF16)
    gs = jnp.dot(oh_s, s_ref[...], preferred_element_type=_F32)
    go = jnp.dot(oh_d, o_ref[...], preferred_element_type=_F32)
    t = gs * go * phr_ref[...]
    af = jnp.sum(t, axis=1) + wb_ref[0, 0]
    weighted = oh_s * af[:, None].astype(_BF16)
    contrib = lax.dot_general(weighted, oh_d, (((0,), (0,)), ((), ())),
                              preferred_element_type=_F32)

    @pl.when(pl.program_id(0) == 0)
    def _():
        atten_ref[...] = jnp.zeros_like(atten_ref)

    atten_ref[...] += contrib


_sc_mesh = plsc.VectorSubcoreMesh(core_axis_name="c", subcore_axis_name="s")


@functools.partial(
    pl.kernel,
    out_type=jax.ShapeDtypeStruct((2, _N * _N), _F32),
    mesh=_sc_mesh,
    scratch_types=[
        pltpu.VMEM((_N, _DG), _F32),
        pltpu.VMEM((_N, _DG), _F32),
        pltpu.VMEM((_CH, _DG), _F32),
        pltpu.VMEM((_CH, _DG), _F32),
        pltpu.SemaphoreType.DMA,
        pltpu.SemaphoreType.DMA,
        pltpu.VMEM((_EPW,), _I32),
        pltpu.VMEM((_EPW,), _I32),
        pltpu.VMEM((_EPW,), _I32),
        pltpu.VMEM((_EPW,), _F32),
        pltpu.VMEM((16,), _F32),
        pltpu.VMEM_SHARED((_N * _N,), _F32),
    ],
    compiler_params=pltpu.CompilerParams(needs_layout_passes=False),
)
def _sc_edge(s_hbm, o_hbm, phr_hbm, src_hbm, dst_hbm, wb_hbm, zeros_hbm,
             out_hbm, s_v, o_v, phr_v0, phr_v1, sem0, sem1, src_v, dst_v,
             idx_v, val_v, wb_v, map_sh):
    c = lax.axis_index("c")
    s = lax.axis_index("s")
    w = s * 2 + c
    eg = w // _NDG
    dg = w % _NDG
    d0 = dg * _DG
    e0 = _E_SC0 + eg * _EPW

    pltpu.sync_copy(s_hbm.at[:, pl.ds(d0, _DG)], s_v)
    pltpu.sync_copy(o_hbm.at[:, pl.ds(d0, _DG)], o_v)
    pltpu.sync_copy(src_hbm.at[pl.ds(e0, _EPW)], src_v)
    pltpu.sync_copy(dst_hbm.at[pl.ds(e0, _EPW)], dst_v)
    pltpu.sync_copy(wb_hbm, wb_v)

    @pl.when(s == 0)
    def _():
        pltpu.sync_copy(zeros_hbm, map_sh)

    plsc.subcore_barrier()

    lanes = lax.iota(_I32, 16)
    wb = wb_v[...]
    base_acc = jnp.where(dg == 0, wb, jnp.zeros((16,), _F32))

    def _phr_start(ch, buf, sem):
        pltpu.async_copy(
            phr_hbm.at[pl.ds(e0 + ch * _CH, _CH), pl.ds(d0, _DG)], buf, sem)

    def _phr_wait(buf, sem):
        pltpu.make_async_copy(phr_hbm.at[pl.ds(e0, _CH), pl.ds(d0, _DG)],
                              buf, sem).wait()

    def _do_chunk(ch, phr_v):
        def _group(g, carry2):
            base = ch * _CH + g * 16
            src_g = src_v[pl.ds(base, 16)]
            dst_g = dst_v[pl.ds(base, 16)]
            lrow = lanes + g * 16

            def _dstep(dd, acc):
                prods = []
                for k in range(8):
                    col = (jnp.full((16,), dd * 8 + k, _I32) + lanes) & (_DG - 1)
                    sv = plsc.load_gather(s_v, [src_g, col])
                    ov = plsc.load_gather(o_v, [dst_g, col])
                    pv = plsc.load_gather(phr_v, [lrow, col])
                    prods.append(sv * ov * pv)
                p01 = prods[0] + prods[1]
                p23 = prods[2] + prods[3]
                p45 = prods[4] + prods[5]
                p67 = prods[6] + prods[7]
                return acc + ((p01 + p23) + (p45 + p67))

            acc = lax.fori_loop(0, _DG // 8, _dstep, base_acc)
            val_v[pl.ds(base, 16)] = acc
            idx_v[pl.ds(base, 16)] = src_g * _N + dst_g
            return carry2

        lax.fori_loop(0, _CH // 16, _group, 0)

    _phr_start(0, phr_v0, sem0)

    def _pair(j, carry):
        ch = j * 2
        _phr_start(ch + 1, phr_v1, sem1)
        _phr_wait(phr_v0, sem0)
        _do_chunk(ch, phr_v0)

        @pl.when(j < _NCH // 2 - 1)
        def _():
            _phr_start(ch + 2, phr_v0, sem0)

        _phr_wait(phr_v1, sem1)
        _do_chunk(ch + 1, phr_v1)
        return carry

    lax.fori_loop(0, _NCH // 2, _pair, 0)

    pltpu.sync_copy(val_v, map_sh.at[idx_v], add=True)
    plsc.subcore_barrier()

    @pl.when(s == 0)
    def _():
        pltpu.sync_copy(map_sh, out_hbm.at[c])


def _finish_body(atten_ref, maps_ref, omega_ref, conv_ref, obj_ref,
                 ln2_g_ref, ln2_b_ref, t1w_ref, t1b_ref, t2w_ref, t2b_ref,
                 out_ref):
    atten = atten_ref[...] + maps_ref[:_N, :] + maps_ref[_N:, :]
    rows = lax.broadcasted_iota(_I32, (_N, _N), 0)
    cols = lax.broadcasted_iota(_I32, (_N, _N), 1)
    a = atten - jnp.where(rows == cols, 10000.0, 0.0)
    m = jnp.max(a, axis=1, keepdims=True)
    ex = jnp.exp(a - m)
    sm = ex / jnp.sum(ex, axis=1, keepdims=True)
    am = omega_ref[...] * sm
    context = _dot3(am, conv_ref[...])
    outputs = obj_ref[...] + context
    mu = jnp.mean(outputs, axis=1, keepdims=True)
    xc = outputs - mu
    var = jnp.mean(xc * xc, axis=1, keepdims=True)
    xn = xc / jnp.sqrt(var + 1e-5) * ln2_g_ref[...] + ln2_b_ref[...]
    h = jax.nn.relu(_dot3(xn, t1w_ref[...]) + t1b_ref[...])
    trans = _dot3(h, t2w_ref[...]) + t2b_ref[...]
    out_ref[...] = jax.nn.relu(outputs + trans)


def _full(shape):
    return pl.BlockSpec(shape, lambda *_: tuple(0 for _ in shape))


def kernel(obj_feats, phr_feats, pair_idxs, ws_w, ws_b, wo_w, wo_b, w_w, w_b,
           conv_w, conv_b, ln1_g, ln1_b, ln2_g, ln2_b,
           trans1_w, trans1_b, trans2_w, trans2_b):
    s_mat, o_mat, s_bf, o_bf, conv_out, omega = pl.pallas_call(
        _prep_body,
        grid=(1,),
        in_specs=[
            _full((_N, _D)), _full((_D, _D)), _full((1, _D)),
            _full((_D, _D)), _full((1, _D)), _full((1, _D)),
            _full((_D, _D)), _full((1, _D)), _full((1, _D)), _full((1, _D)),
        ],
        out_specs=[
            _full((_N, _D)), _full((_N, _D)), _full((_N, _D)),
            _full((_N, _D)), _full((_N, _D)), _full((_N, _N)),
        ],
        out_shape=[
            jax.ShapeDtypeStruct((_N, _D), _F32),
            jax.ShapeDtypeStruct((_N, _D), _F32),
            jax.ShapeDtypeStruct((_N, _D), _BF16),
            jax.ShapeDtypeStruct((_N, _D), _BF16),
            jax.ShapeDtypeStruct((_N, _D), _F32),
            jax.ShapeDtypeStruct((_N, _N), _F32),
        ],
    )(obj_feats, ws_w, ws_b.reshape(1, _D), wo_w, wo_b.reshape(1, _D),
      w_w.reshape(1, _D), conv_w, conv_b.reshape(1, _D),
      ln1_g.reshape(1, _D), ln1_b.reshape(1, _D))

    src = pair_idxs[:, 0]
    dst = pair_idxs[:, 1]
    wb16 = jnp.full((16,), w_b[0], _F32)
    zeros_map = jnp.zeros((_N * _N,), _F32)
    maps = _sc_edge(s_mat, o_mat, phr_feats, src, dst, wb16, zeros_map)

    src3 = src[:_E_SC0].reshape(_NTC, 1, _EB)
    dst3 = dst[:_E_SC0].reshape(_NTC, 1, _EB)
    atten_tc = pl.pallas_call(
        _tc_edge_body,
        grid=(_NTC,),
        in_specs=[
            pl.BlockSpec((1, 1, _EB), lambda i: (i, 0, 0)),
            pl.BlockSpec((1, 1, _EB), lambda i: (i, 0, 0)),
            pl.BlockSpec((_EB, _D), lambda i: (i, 0)),
            pl.BlockSpec((_N, _D), lambda i: (0, 0)),
            pl.BlockSpec((_N, _D), lambda i: (0, 0)),
            pl.BlockSpec(memory_space=pltpu.SMEM),
        ],
        out_specs=pl.BlockSpec((_N, _N), lambda i: (0, 0)),
        out_shape=jax.ShapeDtypeStruct((_N, _N), _F32),
    )(src3, dst3, phr_feats, s_bf, o_bf, w_b.reshape(1, 1))

    return pl.pallas_call(
        _finish_body,
        grid=(1,),
        in_specs=[
            _full((_N, _N)), _full((2 * _N, _N)), _full((_N, _N)),
            _full((_N, _D)), _full((_N, _D)), _full((1, _D)), _full((1, _D)),
            _full((_D, 2 * _D)), _full((1, 2 * _D)),
            _full((2 * _D, _D)), _full((1, _D)),
        ],
        out_specs=_full((_N, _D)),
        out_shape=jax.ShapeDtypeStruct((_N, _D), _F32),
    )(atten_tc, maps.reshape(2 * _N, _N), omega, conv_out, obj_feats,
      ln2_g.reshape(1, _D), ln2_b.reshape(1, _D), trans1_w,
      trans1_b.reshape(1, 2 * _D), trans2_w, trans2_b.reshape(1, _D))
